# baseline (speedup 1.0000x reference)
import functools

import jax
import jax.numpy as jnp
from jax import lax
from jax.experimental import pallas as pl
from jax.experimental.pallas import tpu as pltpu
from jax.experimental.pallas import tpu_sc as plsc

_N = 5000
_BLK = 512
_NB = 10
_NP = _NB * _BLK
_THR = 0.5

_NC = 2
_NS = 16
_NW = _NC * _NS
_CHUNK = _NP // _NW
_RCOLS = 80
_ROWS = _CHUNK * 5 // _RCOLS


def _nms_body(xr, yr, Xr, Yr, xc, yc, Xc, Yc, keep_ref):
    q_lt_p = (lax.broadcasted_iota(jnp.int32, (_BLK, _BLK), 0)
              < lax.broadcasted_iota(jnp.int32, (_BLK, _BLK), 1))

    def block_step(b, carry):
        rx1 = xr[pl.ds(b, 1), :]
        ry1 = yr[pl.ds(b, 1), :]
        rx2 = Xr[pl.ds(b, 1), :]
        ry2 = Yr[pl.ds(b, 1), :]
        r_area = (rx2 - rx1) * (ry2 - ry1)

        def iou_vs(off):
            cx1 = xc[pl.ds(off, _BLK), :]
            cy1 = yc[pl.ds(off, _BLK), :]
            cx2 = Xc[pl.ds(off, _BLK), :]
            cy2 = Yc[pl.ds(off, _BLK), :]
            c_area = (cx2 - cx1) * (cy2 - cy1)
            xx1 = jnp.maximum(cx1, rx1)
            yy1 = jnp.maximum(cy1, ry1)
            xx2 = jnp.minimum(cx2, rx2)
            yy2 = jnp.minimum(cy2, ry2)
            w = jnp.maximum(xx2 - xx1, 0.0)
            h = jnp.maximum(yy2 - yy1, 0.0)
            inter = w * h
            return inter / (c_area + r_area - inter + 1e-9)

        def cross(j, alive):
            adj = (iou_vs(j * _BLK) > _THR).astype(jnp.float32)
            kprev = keep_ref[pl.ds(j, 1), :]
            supp = lax.dot_general(kprev, adj, (((1,), (0,)), ((), ())),
                                   preferred_element_type=jnp.float32)
            return jnp.where(supp > 0.0, 0.0, alive)

        base = lax.fori_loop(0, b, cross, jnp.ones((1, _BLK), jnp.float32))

        adj_self = jnp.where((iou_vs(b * _BLK) > _THR) & q_lt_p, 1.0, 0.0)

        def fix_body(c):
            alive, _ = c
            supp = lax.dot_general(alive, adj_self, (((1,), (0,)), ((), ())),
                                   preferred_element_type=jnp.float32)
            new = jnp.where(supp > 0.0, 0.0, base)
            return new, jnp.any(new != alive)

        alive, _ = lax.while_loop(lambda c: c[1], fix_body, (base, True))
        keep_ref[pl.ds(b, 1), :] = alive
        return carry

    lax.fori_loop(0, _NB, block_step, 0)


def _nms_sorted_keep(bp):
    x, y, X, Y = bp[:, 0], bp[:, 1], bp[:, 2], bp[:, 3]
    args = (x.reshape(_NB, _BLK), y.reshape(_NB, _BLK),
            X.reshape(_NB, _BLK), Y.reshape(_NB, _BLK),
            x.reshape(_NP, 1), y.reshape(_NP, 1),
            X.reshape(_NP, 1), Y.reshape(_NP, 1))
    keep = pl.pallas_call(
        _nms_body,
        out_shape=jax.ShapeDtypeStruct((_NB, _BLK), jnp.float32),
    )(*args)
    return keep.reshape(_NP)


def _sc_scatter_body(xs, ys, Xs, Ys, keep, ordp, sp, out,
                     x_v, y_v, X_v, Y_v, k_v, o_v, s_v, val_m, idx_m, sem):
    wid = lax.axis_index("s") * _NC + lax.axis_index("c")
    base = wid * _CHUNK
    pltpu.sync_copy(xs.at[pl.ds(base, _CHUNK)], x_v)
    pltpu.sync_copy(ys.at[pl.ds(base, _CHUNK)], y_v)
    pltpu.sync_copy(Xs.at[pl.ds(base, _CHUNK)], X_v)
    pltpu.sync_copy(Ys.at[pl.ds(base, _CHUNK)], Y_v)
    pltpu.sync_copy(keep.at[pl.ds(base, _CHUNK)], k_v)
    pltpu.sync_copy(ordp.at[pl.ds(base, _CHUNK)], o_v)
    pltpu.async_copy(sp.at[o_v], s_v, sem).wait()
    for t in range(_CHUNK // 16):
        o16 = o_v[pl.ds(t * 16, 16)]
        k16 = k_v[pl.ds(t * 16, 16)]
        for c, src in enumerate((x_v, y_v, X_v, Y_v, s_v)):
            e0 = c * _CHUNK + t * 16
            r, col = e0 // _RCOLS, e0 % _RCOLS
            val_m[r, pl.ds(col, 16)] = src[pl.ds(t * 16, 16)] * k16
            idx_m[r, pl.ds(col, 16)] = o16 * 5 + c
    copies = [pltpu.async_copy(val_m.at[r], out.at[idx_m.at[r]], sem)
              for r in range(_ROWS)]
    for cp in copies:
        cp.wait()


_sc_scatter = functools.partial(
    pl.kernel,
    out_type=jax.ShapeDtypeStruct((_NP * 5,), jnp.float32),
    mesh=plsc.VectorSubcoreMesh(core_axis_name="c", subcore_axis_name="s"),
    scratch_types=(
        [pltpu.VMEM((_CHUNK,), jnp.float32)] * 5
        + [pltpu.VMEM((_CHUNK,), jnp.int32),
           pltpu.VMEM((_CHUNK,), jnp.float32),
           pltpu.VMEM((_ROWS, _RCOLS), jnp.float32),
           pltpu.VMEM((_ROWS, _RCOLS), jnp.int32),
           pltpu.SemaphoreType.DMA]
    ),
)(_sc_scatter_body)


def kernel(boxes, scores):
    order = jnp.argsort(-scores)
    bs = boxes[order]
    bp = jnp.pad(bs, ((0, _NP - _N), (0, 0)))
    keep_sorted = _nms_sorted_keep(bp)
    x, y, X, Y = bp[:, 0], bp[:, 1], bp[:, 2], bp[:, 3]
    ordp = jnp.concatenate([order, jnp.arange(_N, _NP)]).astype(jnp.int32)
    sp = jnp.pad(scores, (0, _NP - _N))
    outf = _sc_scatter(x, y, X, Y, keep_sorted, ordp, sp)
    return outf[:_N * 5].reshape(_N, 5)


# speedup vs baseline: 1.0285x; 1.0285x over previous
import functools

import jax
import jax.numpy as jnp
from jax import lax
from jax.experimental import pallas as pl
from jax.experimental.pallas import tpu as pltpu
from jax.experimental.pallas import tpu_sc as plsc

_N = 5000
_BLK = 512
_NB = 10
_NP = _NB * _BLK
_THR = 0.5

_NC = 2
_NS = 16
_NW = _NC * _NS
_CHUNK = _NP // _NW
_RCOLS = 80
_ROWS = _CHUNK * 5 // _RCOLS


def _nms_body(xr, yr, Xr, Yr, xc, yc, Xc, Yc, keep_ref):
    q_lt_p = (lax.broadcasted_iota(jnp.int32, (_BLK, _BLK), 0)
              < lax.broadcasted_iota(jnp.int32, (_BLK, _BLK), 1))

    def block_step(b, carry):
        rx1 = xr[pl.ds(b, 1), :]
        ry1 = yr[pl.ds(b, 1), :]
        rx2 = Xr[pl.ds(b, 1), :]
        ry2 = Yr[pl.ds(b, 1), :]
        r_area = (rx2 - rx1) * (ry2 - ry1)

        def iou_vs(off):
            cx1 = xc[pl.ds(off, _BLK), :]
            cy1 = yc[pl.ds(off, _BLK), :]
            cx2 = Xc[pl.ds(off, _BLK), :]
            cy2 = Yc[pl.ds(off, _BLK), :]
            c_area = (cx2 - cx1) * (cy2 - cy1)
            xx1 = jnp.maximum(cx1, rx1)
            yy1 = jnp.maximum(cy1, ry1)
            xx2 = jnp.minimum(cx2, rx2)
            yy2 = jnp.minimum(cy2, ry2)
            w = jnp.maximum(xx2 - xx1, 0.0)
            h = jnp.maximum(yy2 - yy1, 0.0)
            inter = w * h
            return inter / (c_area + r_area - inter + 1e-9)

        def cross(j, alive):
            adj = (iou_vs(j * _BLK) > _THR).astype(jnp.float32)
            kprev = keep_ref[pl.ds(j, 1), :]
            supp = lax.dot_general(kprev, adj, (((1,), (0,)), ((), ())),
                                   preferred_element_type=jnp.float32)
            return jnp.where(supp > 0.0, 0.0, alive)

        base = lax.fori_loop(0, b, cross, jnp.ones((1, _BLK), jnp.float32))

        adj_self = jnp.where((iou_vs(b * _BLK) > _THR) & q_lt_p, 1.0, 0.0)

        def fix_body(c):
            alive, _ = c
            supp = lax.dot_general(alive, adj_self, (((1,), (0,)), ((), ())),
                                   preferred_element_type=jnp.float32)
            new = jnp.where(supp > 0.0, 0.0, base)
            return new, jnp.any(new != alive)

        alive, _ = lax.while_loop(lambda c: c[1], fix_body, (base, True))
        keep_ref[pl.ds(b, 1), :] = alive
        return carry

    lax.fori_loop(0, _NB, block_step, 0)


def _nms_sorted_keep(bp):
    x, y, X, Y = bp[:, 0], bp[:, 1], bp[:, 2], bp[:, 3]
    args = (x.reshape(_NB, _BLK), y.reshape(_NB, _BLK),
            X.reshape(_NB, _BLK), Y.reshape(_NB, _BLK),
            x.reshape(_NP, 1), y.reshape(_NP, 1),
            X.reshape(_NP, 1), Y.reshape(_NP, 1))
    keep = pl.pallas_call(
        _nms_body,
        out_shape=jax.ShapeDtypeStruct((_NB, _BLK), jnp.float32),
    )(*args)
    return keep.reshape(_NP)


def _sc_scatter_body(xs, ys, Xs, Ys, keep, ordp, sp, out,
                     x_v, y_v, X_v, Y_v, k_v, o_v, s_v, val_m, idx_m,
                     sem, sem2):
    wid = lax.axis_index("s") * _NC + lax.axis_index("c")
    base = wid * _CHUNK
    cp_o = pltpu.async_copy(ordp.at[pl.ds(base, _CHUNK)], o_v, sem2)
    stage = [pltpu.async_copy(src.at[pl.ds(base, _CHUNK)], dst, sem)
             for src, dst in ((xs, x_v), (ys, y_v), (Xs, X_v), (Ys, Y_v),
                              (keep, k_v))]
    cp_o.wait()
    cp_s = pltpu.async_copy(sp.at[o_v], s_v, sem2)
    for cp in stage:
        cp.wait()
    cp_s.wait()
    for t in range(_CHUNK // 16):
        o16 = o_v[pl.ds(t * 16, 16)]
        k16 = k_v[pl.ds(t * 16, 16)]
        for c, src in enumerate((x_v, y_v, X_v, Y_v, s_v)):
            e0 = c * _CHUNK + t * 16
            r, col = e0 // _RCOLS, e0 % _RCOLS
            val_m[r, pl.ds(col, 16)] = src[pl.ds(t * 16, 16)] * k16
            idx_m[r, pl.ds(col, 16)] = o16 * 5 + c
    copies = [pltpu.async_copy(val_m.at[r], out.at[idx_m.at[r]], sem)
              for r in range(_ROWS)]
    for cp in copies:
        cp.wait()


_sc_scatter = functools.partial(
    pl.kernel,
    out_type=jax.ShapeDtypeStruct((_NP * 5,), jnp.float32),
    mesh=plsc.VectorSubcoreMesh(core_axis_name="c", subcore_axis_name="s"),
    scratch_types=(
        [pltpu.VMEM((_CHUNK,), jnp.float32)] * 5
        + [pltpu.VMEM((_CHUNK,), jnp.int32),
           pltpu.VMEM((_CHUNK,), jnp.float32),
           pltpu.VMEM((_ROWS, _RCOLS), jnp.float32),
           pltpu.VMEM((_ROWS, _RCOLS), jnp.int32),
           pltpu.SemaphoreType.DMA,
           pltpu.SemaphoreType.DMA]
    ),
)(_sc_scatter_body)


def kernel(boxes, scores):
    order = jnp.argsort(-scores)
    bs = boxes[order]
    bp = jnp.pad(bs, ((0, _NP - _N), (0, 0)))
    keep_sorted = _nms_sorted_keep(bp)
    x, y, X, Y = bp[:, 0], bp[:, 1], bp[:, 2], bp[:, 3]
    ordp = jnp.concatenate([order, jnp.arange(_N, _NP)]).astype(jnp.int32)
    sp = jnp.pad(scores, (0, _NP - _N))
    outf = _sc_scatter(x, y, X, Y, keep_sorted, ordp, sp)
    return outf[:_N * 5].reshape(_N, 5)


# speedup vs baseline: 1.3088x; 1.2726x over previous
import functools

import jax
import jax.numpy as jnp
from jax import lax
from jax.experimental import pallas as pl
from jax.experimental.pallas import tpu as pltpu
from jax.experimental.pallas import tpu_sc as plsc

_N = 5000
_BLK = 512
_NB = 10
_NP = _NB * _BLK
_THR = 0.5

_NC = 2
_NS = 16
_NW = _NC * _NS
_CHUNK = _NP // _NW
_RCOLS = 80
_ROWS = _CHUNK * 5 // _RCOLS


def _nms_body(xr, yr, Xr, Yr, xc, yc, Xc, Yc, keep_ref):
    q_lt_p = (lax.broadcasted_iota(jnp.int32, (_BLK, _BLK), 0)
              < lax.broadcasted_iota(jnp.int32, (_BLK, _BLK), 1))

    def block_step(b, carry):
        rx1 = xr[pl.ds(b, 1), :]
        ry1 = yr[pl.ds(b, 1), :]
        rx2 = Xr[pl.ds(b, 1), :]
        ry2 = Yr[pl.ds(b, 1), :]
        r_area = (rx2 - rx1) * (ry2 - ry1)

        def iou_vs(off):
            cx1 = xc[pl.ds(off, _BLK), :]
            cy1 = yc[pl.ds(off, _BLK), :]
            cx2 = Xc[pl.ds(off, _BLK), :]
            cy2 = Yc[pl.ds(off, _BLK), :]
            c_area = (cx2 - cx1) * (cy2 - cy1)
            xx1 = jnp.maximum(cx1, rx1)
            yy1 = jnp.maximum(cy1, ry1)
            xx2 = jnp.minimum(cx2, rx2)
            yy2 = jnp.minimum(cy2, ry2)
            w = jnp.maximum(xx2 - xx1, 0.0)
            h = jnp.maximum(yy2 - yy1, 0.0)
            inter = w * h
            return inter / (c_area + r_area - inter + 1e-9)

        def cross(j, alive):
            adj = (iou_vs(j * _BLK) > _THR).astype(jnp.float32)
            kprev = keep_ref[pl.ds(j, 1), :]
            supp = lax.dot_general(kprev, adj, (((1,), (0,)), ((), ())),
                                   preferred_element_type=jnp.float32)
            return jnp.where(supp > 0.0, 0.0, alive)

        base = lax.fori_loop(0, b, cross, jnp.ones((1, _BLK), jnp.float32))

        adj_self = jnp.where((iou_vs(b * _BLK) > _THR) & q_lt_p, 1.0, 0.0)

        def fix_body(c):
            alive, _ = c
            supp = lax.dot_general(alive, adj_self, (((1,), (0,)), ((), ())),
                                   preferred_element_type=jnp.float32)
            new = jnp.where(supp > 0.0, 0.0, base)
            return new, jnp.any(new != alive)

        alive, _ = lax.while_loop(lambda c: c[1], fix_body, (base, True))
        keep_ref[pl.ds(b, 1), :] = alive
        return carry

    lax.fori_loop(0, _NB, block_step, 0)


def _nms_sorted_keep(bp):
    x, y, X, Y = bp[:, 0], bp[:, 1], bp[:, 2], bp[:, 3]
    args = (x.reshape(_NB, _BLK), y.reshape(_NB, _BLK),
            X.reshape(_NB, _BLK), Y.reshape(_NB, _BLK),
            x.reshape(_NP, 1), y.reshape(_NP, 1),
            X.reshape(_NP, 1), Y.reshape(_NP, 1))
    keep = pl.pallas_call(
        _nms_body,
        out_shape=jax.ShapeDtypeStruct((_NB, _BLK), jnp.float32),
    )(*args)
    return keep.reshape(_NP)


def _sc_mask_body(keep, ordp, out, k_v, o_v, sem, sem2):
    wid = lax.axis_index("s") * _NC + lax.axis_index("c")
    base = wid * _CHUNK
    cp_o = pltpu.async_copy(ordp.at[pl.ds(base, _CHUNK)], o_v, sem2)
    cp_k = pltpu.async_copy(keep.at[pl.ds(base, _CHUNK)], k_v, sem)
    cp_o.wait()
    cp_k.wait()
    pltpu.async_copy(k_v, out.at[o_v], sem).wait()


_sc_mask = functools.partial(
    pl.kernel,
    out_type=jax.ShapeDtypeStruct((_NP,), jnp.float32),
    mesh=plsc.VectorSubcoreMesh(core_axis_name="c", subcore_axis_name="s"),
    scratch_types=[
        pltpu.VMEM((_CHUNK,), jnp.float32),
        pltpu.VMEM((_CHUNK,), jnp.int32),
        pltpu.SemaphoreType.DMA,
        pltpu.SemaphoreType.DMA,
    ],
)(_sc_mask_body)


def kernel(boxes, scores):
    order = jnp.argsort(-scores)
    bs = boxes[order]
    bp = jnp.pad(bs, ((0, _NP - _N), (0, 0)))
    keep_sorted = _nms_sorted_keep(bp)
    ordp = jnp.concatenate([order, jnp.arange(_N, _NP)]).astype(jnp.int32)
    mask = _sc_mask(keep_sorted, ordp)[:_N]
    out = jnp.concatenate([boxes * mask[:, None], (scores * mask)[:, None]],
                          axis=1)
    return out


# speedup vs baseline: 1.4527x; 1.1099x over previous
import functools

import jax
import jax.numpy as jnp
from jax import lax
from jax.experimental import pallas as pl
from jax.experimental.pallas import tpu as pltpu
from jax.experimental.pallas import tpu_sc as plsc

_N = 5000
_BLK = 1024
_NB = 5
_NP = _NB * _BLK
_THR = 0.5

_NC = 2
_NS = 16
_NW = _NC * _NS
_CHUNK = _NP // _NW
_RCOLS = 80
_ROWS = _CHUNK * 5 // _RCOLS


def _nms_body(xr, yr, Xr, Yr, xc, yc, Xc, Yc, keep_ref):
    q_lt_p = (lax.broadcasted_iota(jnp.int32, (_BLK, _BLK), 0)
              < lax.broadcasted_iota(jnp.int32, (_BLK, _BLK), 1))

    def block_step(b, carry):
        rx1 = xr[pl.ds(b, 1), :]
        ry1 = yr[pl.ds(b, 1), :]
        rx2 = Xr[pl.ds(b, 1), :]
        ry2 = Yr[pl.ds(b, 1), :]
        r_area = (rx2 - rx1) * (ry2 - ry1)

        def iou_vs(off):
            cx1 = xc[pl.ds(off, _BLK), :]
            cy1 = yc[pl.ds(off, _BLK), :]
            cx2 = Xc[pl.ds(off, _BLK), :]
            cy2 = Yc[pl.ds(off, _BLK), :]
            c_area = (cx2 - cx1) * (cy2 - cy1)
            xx1 = jnp.maximum(cx1, rx1)
            yy1 = jnp.maximum(cy1, ry1)
            xx2 = jnp.minimum(cx2, rx2)
            yy2 = jnp.minimum(cy2, ry2)
            w = jnp.maximum(xx2 - xx1, 0.0)
            h = jnp.maximum(yy2 - yy1, 0.0)
            inter = w * h
            return inter / (c_area + r_area - inter + 1e-9)

        def cross(j, alive):
            adj = (iou_vs(j * _BLK) > _THR).astype(jnp.float32)
            kprev = keep_ref[pl.ds(j, 1), :]
            supp = lax.dot_general(kprev, adj, (((1,), (0,)), ((), ())),
                                   preferred_element_type=jnp.float32)
            return jnp.where(supp > 0.0, 0.0, alive)

        base = lax.fori_loop(0, b, cross, jnp.ones((1, _BLK), jnp.float32))

        adj_self = jnp.where((iou_vs(b * _BLK) > _THR) & q_lt_p, 1.0, 0.0)

        def fix_body(c):
            alive, _ = c
            supp = lax.dot_general(alive, adj_self, (((1,), (0,)), ((), ())),
                                   preferred_element_type=jnp.float32)
            new = jnp.where(supp > 0.0, 0.0, base)
            return new, jnp.any(new != alive)

        alive, _ = lax.while_loop(lambda c: c[1], fix_body, (base, True))
        keep_ref[pl.ds(b, 1), :] = alive
        return carry

    lax.fori_loop(0, _NB, block_step, 0)


def _nms_sorted_keep(bp):
    x, y, X, Y = bp[:, 0], bp[:, 1], bp[:, 2], bp[:, 3]
    args = (x.reshape(_NB, _BLK), y.reshape(_NB, _BLK),
            X.reshape(_NB, _BLK), Y.reshape(_NB, _BLK),
            x.reshape(_NP, 1), y.reshape(_NP, 1),
            X.reshape(_NP, 1), Y.reshape(_NP, 1))
    keep = pl.pallas_call(
        _nms_body,
        out_shape=jax.ShapeDtypeStruct((_NB, _BLK), jnp.float32),
    )(*args)
    return keep.reshape(_NP)


def _sc_mask_body(keep, ordp, out, k_v, o_v, sem, sem2):
    wid = lax.axis_index("s") * _NC + lax.axis_index("c")
    base = wid * _CHUNK
    cp_o = pltpu.async_copy(ordp.at[pl.ds(base, _CHUNK)], o_v, sem2)
    cp_k = pltpu.async_copy(keep.at[pl.ds(base, _CHUNK)], k_v, sem)
    cp_o.wait()
    cp_k.wait()
    pltpu.async_copy(k_v, out.at[o_v], sem).wait()


_sc_mask = functools.partial(
    pl.kernel,
    out_type=jax.ShapeDtypeStruct((_NP,), jnp.float32),
    mesh=plsc.VectorSubcoreMesh(core_axis_name="c", subcore_axis_name="s"),
    scratch_types=[
        pltpu.VMEM((_CHUNK,), jnp.float32),
        pltpu.VMEM((_CHUNK,), jnp.int32),
        pltpu.SemaphoreType.DMA,
        pltpu.SemaphoreType.DMA,
    ],
)(_sc_mask_body)


def kernel(boxes, scores):
    order = jnp.argsort(-scores)
    bs = boxes[order]
    bp = jnp.pad(bs, ((0, _NP - _N), (0, 0)))
    keep_sorted = _nms_sorted_keep(bp)
    ordp = jnp.concatenate([order, jnp.arange(_N, _NP)]).astype(jnp.int32)
    mask = _sc_mask(keep_sorted, ordp)[:_N]
    out = jnp.concatenate([boxes * mask[:, None], (scores * mask)[:, None]],
                          axis=1)
    return out


# speedup vs baseline: 1.4744x; 1.0149x over previous
import functools

import jax
import jax.numpy as jnp
from jax import lax
from jax.experimental import pallas as pl
from jax.experimental.pallas import tpu as pltpu
from jax.experimental.pallas import tpu_sc as plsc

_N = 5000
_BLK = 1280
_NB = 4
_NP = _NB * _BLK
_THR = 0.5

_NC = 2
_NS = 16
_NW = _NC * _NS
_CHUNK = _NP // _NW
_RCOLS = 80
_ROWS = _CHUNK * 5 // _RCOLS


def _nms_body(xr, yr, Xr, Yr, xc, yc, Xc, Yc, keep_ref):
    q_lt_p = (lax.broadcasted_iota(jnp.int32, (_BLK, _BLK), 0)
              < lax.broadcasted_iota(jnp.int32, (_BLK, _BLK), 1))

    def block_step(b, carry):
        rx1 = xr[pl.ds(b, 1), :]
        ry1 = yr[pl.ds(b, 1), :]
        rx2 = Xr[pl.ds(b, 1), :]
        ry2 = Yr[pl.ds(b, 1), :]
        r_area = (rx2 - rx1) * (ry2 - ry1)

        def iou_vs(off):
            cx1 = xc[pl.ds(off, _BLK), :]
            cy1 = yc[pl.ds(off, _BLK), :]
            cx2 = Xc[pl.ds(off, _BLK), :]
            cy2 = Yc[pl.ds(off, _BLK), :]
            c_area = (cx2 - cx1) * (cy2 - cy1)
            xx1 = jnp.maximum(cx1, rx1)
            yy1 = jnp.maximum(cy1, ry1)
            xx2 = jnp.minimum(cx2, rx2)
            yy2 = jnp.minimum(cy2, ry2)
            w = jnp.maximum(xx2 - xx1, 0.0)
            h = jnp.maximum(yy2 - yy1, 0.0)
            inter = w * h
            return inter / (c_area + r_area - inter + 1e-9)

        def cross(j, alive):
            adj = (iou_vs(j * _BLK) > _THR).astype(jnp.float32)
            kprev = keep_ref[pl.ds(j, 1), :]
            supp = lax.dot_general(kprev, adj, (((1,), (0,)), ((), ())),
                                   preferred_element_type=jnp.float32)
            return jnp.where(supp > 0.0, 0.0, alive)

        base = lax.fori_loop(0, b, cross, jnp.ones((1, _BLK), jnp.float32))

        adj_self = jnp.where((iou_vs(b * _BLK) > _THR) & q_lt_p, 1.0, 0.0)

        def fix_body(c):
            alive, _ = c
            supp = lax.dot_general(alive, adj_self, (((1,), (0,)), ((), ())),
                                   preferred_element_type=jnp.float32)
            new = jnp.where(supp > 0.0, 0.0, base)
            return new, jnp.any(new != alive)

        alive, _ = lax.while_loop(lambda c: c[1], fix_body, (base, True))
        keep_ref[pl.ds(b, 1), :] = alive
        return carry

    lax.fori_loop(0, _NB, block_step, 0)


def _nms_sorted_keep(bp):
    x, y, X, Y = bp[:, 0], bp[:, 1], bp[:, 2], bp[:, 3]
    args = (x.reshape(_NB, _BLK), y.reshape(_NB, _BLK),
            X.reshape(_NB, _BLK), Y.reshape(_NB, _BLK),
            x.reshape(_NP, 1), y.reshape(_NP, 1),
            X.reshape(_NP, 1), Y.reshape(_NP, 1))
    keep = pl.pallas_call(
        _nms_body,
        out_shape=jax.ShapeDtypeStruct((_NB, _BLK), jnp.float32),
    )(*args)
    return keep.reshape(_NP)


def _sc_mask_body(keep, ordp, out, k_v, o_v, sem, sem2):
    wid = lax.axis_index("s") * _NC + lax.axis_index("c")
    base = wid * _CHUNK
    cp_o = pltpu.async_copy(ordp.at[pl.ds(base, _CHUNK)], o_v, sem2)
    cp_k = pltpu.async_copy(keep.at[pl.ds(base, _CHUNK)], k_v, sem)
    cp_o.wait()
    cp_k.wait()
    pltpu.async_copy(k_v, out.at[o_v], sem).wait()


_sc_mask = functools.partial(
    pl.kernel,
    out_type=jax.ShapeDtypeStruct((_NP,), jnp.float32),
    mesh=plsc.VectorSubcoreMesh(core_axis_name="c", subcore_axis_name="s"),
    scratch_types=[
        pltpu.VMEM((_CHUNK,), jnp.float32),
        pltpu.VMEM((_CHUNK,), jnp.int32),
        pltpu.SemaphoreType.DMA,
        pltpu.SemaphoreType.DMA,
    ],
)(_sc_mask_body)


def kernel(boxes, scores):
    order = jnp.argsort(-scores)
    bs = boxes[order]
    bp = jnp.pad(bs, ((0, _NP - _N), (0, 0)))
    keep_sorted = _nms_sorted_keep(bp)
    ordp = jnp.concatenate([order, jnp.arange(_N, _NP)]).astype(jnp.int32)
    mask = _sc_mask(keep_sorted, ordp)[:_N]
    out = jnp.concatenate([boxes * mask[:, None], (scores * mask)[:, None]],
                          axis=1)
    return out
